# baseline (speedup 1.0000x reference)
import functools

import jax
import jax.numpy as jnp
from jax import lax
from jax.experimental import pallas as pl
from jax.experimental.pallas import tpu as pltpu
from jax.experimental.pallas import tpu_sc as plsc

N_USERS = 100000
EMBED_DIM = 64
B = 4096

NUM_CORES = 2
NUM_SUBCORES = 16
LANES = 16
NW = NUM_CORES * NUM_SUBCORES
PER_W = B // NW
ROWS_PER_W = 2 * PER_W

S4 = 25088
TROW = 6272
TGRID = S4 // TROW
MLP_GRID = 4


def _sc_gather_body(users_hbm, items_hbm, table_hbm, out_hbm,
                    idx_v, rows_v, sem):
    wid = lax.axis_index("s") * NUM_CORES + lax.axis_index("c")
    base = wid * PER_W
    pltpu.sync_copy(users_hbm.at[pl.ds(base, PER_W)], idx_v.at[pl.ds(0, PER_W)])
    pltpu.sync_copy(items_hbm.at[pl.ds(base, PER_W)],
                    idx_v.at[pl.ds(PER_W, PER_W)])
    lane = lax.iota(jnp.int32, LANES)

    def chunk_body(c, _):
        vec = idx_v[pl.ds(c * LANES, LANES)]
        vec = vec - jnp.where(vec >= 2 * S4, 2 * S4, 0)
        vec = vec - jnp.where(vec >= S4, S4, 0)
        for j in range(LANES):
            p = jnp.sum(jnp.where(lane == j, vec, 0))
            pltpu.async_copy(table_hbm.at[pl.ds(p, 1)],
                             rows_v.at[pl.ds(c * LANES + j, 1)], sem)
        return 0

    lax.fori_loop(0, ROWS_PER_W // LANES, chunk_body, 0)
    pltpu.make_async_copy(table_hbm.at[pl.ds(0, ROWS_PER_W)], rows_v, sem).wait()
    pltpu.sync_copy(rows_v.at[pl.ds(0, PER_W)], out_hbm.at[pl.ds(base, PER_W)])
    pltpu.sync_copy(rows_v.at[pl.ds(PER_W, PER_W)],
                    out_hbm.at[pl.ds(B + base, PER_W)])


def _make_sc_gather():
    return functools.partial(
        pl.kernel,
        mesh=plsc.VectorSubcoreMesh(core_axis_name="c", subcore_axis_name="s"),
        out_type=jax.ShapeDtypeStruct((2 * B, 2 * EMBED_DIM), jnp.int32),
        scratch_types=[
            pltpu.VMEM((ROWS_PER_W,), jnp.int32),
            pltpu.VMEM((ROWS_PER_W, 2 * EMBED_DIM), jnp.int32),
            pltpu.SemaphoreType.DMA,
        ],
        compiler_params=pltpu.CompilerParams(needs_layout_passes=False),
    )(_sc_gather_body)


def _unpack(w_ref, idx_ref):
    idx = idx_ref[...]
    ge1 = idx >= S4
    ge2 = idx >= 2 * S4
    ge3 = idx >= 3 * S4
    m = ge1 ^ ge2 ^ ge3
    w = jnp.where(m, w_ref[:, EMBED_DIM:], w_ref[:, :EMBED_DIM])
    wu = lax.bitcast_convert_type(w, jnp.uint32)
    lo = lax.bitcast_convert_type(wu << 16, jnp.float32)
    hi = lax.bitcast_convert_type(wu & jnp.uint32(0xFFFF0000), jnp.float32)
    return jnp.where(ge2, hi, lo)


def _bmm(a, b):
    return lax.dot(a.astype(jnp.bfloat16), b.astype(jnp.bfloat16),
                   preferred_element_type=jnp.float32)


def _mlp_body(u_ref, i_ref, gu_ref, gi_ref, w1_ref, b1_ref, w2_ref, b2_ref,
              w3_ref, b3_ref, w4_ref, out_ref):
    u = _unpack(u_ref, gu_ref)
    it = _unpack(i_ref, gi_ref)
    x = (_bmm(u, w1_ref[:EMBED_DIM, :]) + _bmm(it, w1_ref[EMBED_DIM:, :])
         + b1_ref[...].reshape(1, -1))
    x = jnp.maximum(x, 0.0)
    x = jnp.maximum(_bmm(x, w2_ref[...]) + b2_ref[...].reshape(1, -1), 0.0)
    x = jnp.maximum(_bmm(x, w3_ref[...]) + b3_ref[...].reshape(1, -1), 0.0)
    out_ref[...] = x @ w4_ref[...]


def _rtne_bf16_hi(x):
    b = lax.bitcast_convert_type(x, jnp.uint32)
    r = b + jnp.uint32(0x7FFF) + ((b >> 16) & jnp.uint32(1))
    return r & jnp.uint32(0xFFFF0000)


def _transpose_body(t0_ref, t1_ref, t2_ref, t3_ref, out_ref):
    n = 2 * EMBED_DIM
    row = lax.broadcasted_iota(jnp.int32, (n, n), 0)
    col = lax.broadcasted_iota(jnp.int32, (n, n), 1)
    eye = (row == col).astype(jnp.float32)
    a01 = jnp.concatenate([t0_ref[...], t1_ref[...]], axis=0)
    t3 = t3_ref[...]
    t3 = jnp.where(jnp.abs(t3) < jnp.float32(1e30), t3, 0.0)
    a23 = jnp.concatenate([t2_ref[...], t3], axis=0)
    ta = lax.dot_general(a01, eye, (((0,), (0,)), ((), ())),
                         preferred_element_type=jnp.float32)
    tb = lax.dot_general(a23, eye, (((0,), (0,)), ((), ())),
                         preferred_element_type=jnp.float32)
    packed = (_rtne_bf16_hi(ta) >> 16) | _rtne_bf16_hi(tb)
    out_ref[...] = lax.bitcast_convert_type(packed, jnp.int32)


def _relayout_table(table_user):
    tt = table_user.T
    return pl.pallas_call(
        _transpose_body,
        grid=(TGRID,),
        in_specs=[
            pl.BlockSpec((EMBED_DIM, TROW), lambda i: (0, i)),
            pl.BlockSpec((EMBED_DIM, TROW), lambda i: (0, i + TGRID)),
            pl.BlockSpec((EMBED_DIM, TROW), lambda i: (0, i + 2 * TGRID)),
            pl.BlockSpec((EMBED_DIM, TROW), lambda i: (0, i + 3 * TGRID)),
        ],
        out_specs=pl.BlockSpec((TROW, 2 * EMBED_DIM), lambda i: (i, 0)),
        out_shape=jax.ShapeDtypeStruct((S4, 2 * EMBED_DIM), jnp.int32),
        compiler_params=pltpu.CompilerParams(fuse_transposed_lhs_in_matmul=True),
    )(tt, tt, tt, tt)


def kernel(users, items, table_user, W1, b1, W2, b2, W3, b3, W4):
    users = users.astype(jnp.int32)
    items = items.astype(jnp.int32)
    tbl = _relayout_table(table_user)
    emb = _make_sc_gather()(users, items, tbl)
    bm = B // MLP_GRID
    nb = B // bm
    full = lambda a: pl.BlockSpec(a.shape, lambda i: (0,) * a.ndim)
    score = pl.pallas_call(
        _mlp_body,
        grid=(MLP_GRID,),
        in_specs=[
            pl.BlockSpec((bm, 2 * EMBED_DIM), lambda i: (i, 0)),
            pl.BlockSpec((bm, 2 * EMBED_DIM), lambda i: (i + nb, 0)),
            pl.BlockSpec((bm, 1), lambda i: (i, 0)),
            pl.BlockSpec((bm, 1), lambda i: (i, 0)),
            full(W1), full(b1), full(W2), full(b2), full(W3), full(b3),
            full(W4),
        ],
        out_specs=pl.BlockSpec((bm, 1), lambda i: (i, 0)),
        out_shape=jax.ShapeDtypeStruct((B, 1), jnp.float32),
    )(emb, emb, users.reshape(B, 1), items.reshape(B, 1),
      W1, b1, W2, b2, W3, b3, W4)
    return score


# speedup vs baseline: 1.0696x; 1.0696x over previous
import functools

import jax
import jax.numpy as jnp
from jax import lax
from jax.experimental import pallas as pl
from jax.experimental.pallas import tpu as pltpu
from jax.experimental.pallas import tpu_sc as plsc

N_USERS = 100000
EMBED_DIM = 64
B = 4096

NUM_CORES = 2
NUM_SUBCORES = 16
LANES = 16
NW = NUM_CORES * NUM_SUBCORES
PER_W = B // NW
ROWS_PER_W = 2 * PER_W

S4 = 25088
TROW = 6272
TGRID = S4 // TROW
MLP_GRID = 4


def _sc_gather_body(users_hbm, items_hbm, table_hbm, out_hbm,
                    idx_v, rows_v, sem):
    wid = lax.axis_index("s") * NUM_CORES + lax.axis_index("c")
    base = wid * PER_W
    pltpu.sync_copy(users_hbm.at[pl.ds(base, PER_W)], idx_v.at[pl.ds(0, PER_W)])
    pltpu.sync_copy(items_hbm.at[pl.ds(base, PER_W)],
                    idx_v.at[pl.ds(PER_W, PER_W)])
    lane = lax.iota(jnp.int32, LANES)

    def chunk_body(c, _):
        vec = idx_v[pl.ds(c * LANES, LANES)]
        vec = vec - jnp.where(vec >= 2 * S4, 2 * S4, 0)
        vec = vec - jnp.where(vec >= S4, S4, 0)
        for j in range(LANES):
            p = jnp.sum(jnp.where(lane == j, vec, 0))
            pltpu.async_copy(table_hbm.at[pl.ds(p, 1)],
                             rows_v.at[pl.ds(c * LANES + j, 1)], sem)
        return 0

    lax.fori_loop(0, ROWS_PER_W // LANES, chunk_body, 0)
    pltpu.make_async_copy(table_hbm.at[pl.ds(0, ROWS_PER_W)], rows_v, sem).wait()
    pltpu.sync_copy(rows_v.at[pl.ds(0, PER_W)], out_hbm.at[pl.ds(base, PER_W)])
    pltpu.sync_copy(rows_v.at[pl.ds(PER_W, PER_W)],
                    out_hbm.at[pl.ds(B + base, PER_W)])


def _make_sc_gather():
    return functools.partial(
        pl.kernel,
        mesh=plsc.VectorSubcoreMesh(core_axis_name="c", subcore_axis_name="s"),
        out_type=jax.ShapeDtypeStruct((2 * B, 2 * EMBED_DIM), jnp.int32),
        scratch_types=[
            pltpu.VMEM((ROWS_PER_W,), jnp.int32),
            pltpu.VMEM((ROWS_PER_W, 2 * EMBED_DIM), jnp.int32),
            pltpu.SemaphoreType.DMA,
        ],
        compiler_params=pltpu.CompilerParams(needs_layout_passes=False),
    )(_sc_gather_body)


def _unpack(w_ref, idx_ref):
    idx = idx_ref[...]
    ge1 = idx >= S4
    ge2 = idx >= 2 * S4
    ge3 = idx >= 3 * S4
    m = ge1 ^ ge2 ^ ge3
    w = jnp.where(m, w_ref[:, EMBED_DIM:], w_ref[:, :EMBED_DIM])
    wu = lax.bitcast_convert_type(w, jnp.uint32)
    lo = lax.bitcast_convert_type(wu << 16, jnp.float32)
    hi = lax.bitcast_convert_type(wu & jnp.uint32(0xFFFF0000), jnp.float32)
    return jnp.where(ge2, hi, lo)


def _mlp_body(u_ref, i_ref, gu_ref, gi_ref, w1_ref, b1_ref, w2_ref, b2_ref,
              w3_ref, b3_ref, w4_ref, out_ref):
    u = _unpack(u_ref, gu_ref)
    it = _unpack(i_ref, gi_ref)
    x = (u @ w1_ref[:EMBED_DIM, :] + it @ w1_ref[EMBED_DIM:, :]
         + b1_ref[...].reshape(1, -1))
    x = jnp.maximum(x, 0.0)
    x = jnp.maximum(x @ w2_ref[...] + b2_ref[...].reshape(1, -1), 0.0)
    x = jnp.maximum(x @ w3_ref[...] + b3_ref[...].reshape(1, -1), 0.0)
    out_ref[...] = (x @ w4_ref[...]).T


def _rtne_bf16_hi(x):
    b = lax.bitcast_convert_type(x, jnp.uint32)
    r = b + jnp.uint32(0x7FFF) + ((b >> 16) & jnp.uint32(1))
    return r & jnp.uint32(0xFFFF0000)


def _transpose_body(t0_ref, t1_ref, t2_ref, t3_ref, out_ref):
    n = 2 * EMBED_DIM
    row = lax.broadcasted_iota(jnp.int32, (n, n), 0)
    col = lax.broadcasted_iota(jnp.int32, (n, n), 1)
    eye = (row == col).astype(jnp.float32)
    a01 = jnp.concatenate([t0_ref[...], t1_ref[...]], axis=0)
    t3 = t3_ref[...]
    t3 = jnp.where(jnp.abs(t3) < jnp.float32(1e30), t3, 0.0)
    a23 = jnp.concatenate([t2_ref[...], t3], axis=0)
    ta = lax.dot_general(a01, eye, (((0,), (0,)), ((), ())),
                         preferred_element_type=jnp.float32)
    tb = lax.dot_general(a23, eye, (((0,), (0,)), ((), ())),
                         preferred_element_type=jnp.float32)
    packed = (_rtne_bf16_hi(ta) >> 16) | _rtne_bf16_hi(tb)
    out_ref[...] = lax.bitcast_convert_type(packed, jnp.int32)


def _relayout_table(table_user):
    tt = table_user.T
    return pl.pallas_call(
        _transpose_body,
        grid=(TGRID,),
        in_specs=[
            pl.BlockSpec((EMBED_DIM, TROW), lambda i: (0, i)),
            pl.BlockSpec((EMBED_DIM, TROW), lambda i: (0, i + TGRID)),
            pl.BlockSpec((EMBED_DIM, TROW), lambda i: (0, i + 2 * TGRID)),
            pl.BlockSpec((EMBED_DIM, TROW), lambda i: (0, i + 3 * TGRID)),
        ],
        out_specs=pl.BlockSpec((TROW, 2 * EMBED_DIM), lambda i: (i, 0)),
        out_shape=jax.ShapeDtypeStruct((S4, 2 * EMBED_DIM), jnp.int32),
        compiler_params=pltpu.CompilerParams(fuse_transposed_lhs_in_matmul=True),
    )(tt, tt, tt, tt)


def kernel(users, items, table_user, W1, b1, W2, b2, W3, b3, W4):
    users = users.astype(jnp.int32)
    items = items.astype(jnp.int32)
    tbl = _relayout_table(table_user)
    emb = _make_sc_gather()(users, items, tbl)
    bm = B // MLP_GRID
    nb = B // bm
    full = lambda a: pl.BlockSpec(a.shape, lambda i: (0,) * a.ndim)
    score = pl.pallas_call(
        _mlp_body,
        grid=(MLP_GRID,),
        in_specs=[
            pl.BlockSpec((bm, 2 * EMBED_DIM), lambda i: (i, 0)),
            pl.BlockSpec((bm, 2 * EMBED_DIM), lambda i: (i + nb, 0)),
            pl.BlockSpec((bm, 1), lambda i: (i, 0)),
            pl.BlockSpec((bm, 1), lambda i: (i, 0)),
            full(W1), full(b1), full(W2), full(b2), full(W3), full(b3),
            full(W4),
        ],
        out_specs=pl.BlockSpec((1, bm), lambda i: (0, i)),
        out_shape=jax.ShapeDtypeStruct((1, B), jnp.float32),
    )(emb, emb, users.reshape(B, 1), items.reshape(B, 1),
      W1, b1, W2, b2, W3, b3, W4)
    return score.T
